# Initial kernel scaffold; baseline (speedup 1.0000x reference)
import jax
import jax.numpy as jnp
from jax.experimental import pallas as pl


def kernel(t_cls_0, t_bbox_0, t_angle_0, t_ctr_0, t_cls_1, t_bbox_1, t_angle_1, t_ctr_1, t_cls_2, t_bbox_2, t_angle_2, t_ctr_2, t_cls_3, t_bbox_3, t_angle_3, t_ctr_3, t_cls_4, t_bbox_4, t_angle_4, t_ctr_4, s_cls_0, s_bbox_0, s_angle_0, s_ctr_0, s_cls_1, s_bbox_1, s_angle_1, s_ctr_1, s_cls_2, s_bbox_2, s_angle_2, s_ctr_2, s_cls_3, s_bbox_3, s_angle_3, s_ctr_3, s_cls_4, s_bbox_4, s_angle_4, s_ctr_4):
    raise NotImplementedError("write your pallas kernel here")


# speedup vs baseline: 2.3323x; 2.3323x over previous
import jax
import jax.numpy as jnp
from jax.experimental import pallas as pl
from jax.experimental.pallas import tpu as pltpu

_B = 16
_CLS = 16
_HS = (64, 32, 16, 8, 4)
_HH = tuple(h * h for h in _HS)
_NPB = sum(_HH)
_N = _B * _NPB
_K = max(int(_N * 0.01), 2)
_OFF = (0, 4096, 5120, 5376, 5440)
_NLVL = 5
_ONE_BITS = 0x3F800001


def _bce(p, t):
    p = jnp.clip(p, 1e-12, 1.0 - 1e-12)
    return -(t * jnp.log(p) + (1.0 - t) * jnp.log1p(-p))


def _smooth_l1(x, y):
    d = jnp.abs(x - y)
    return jnp.where(d < 1.0, 0.5 * d * d, d - 0.5)


def _anchor_index(shape, lvl):
    b = jax.lax.broadcasted_iota(jnp.int32, shape, 0)
    p = jax.lax.broadcasted_iota(jnp.int32, shape, 2)
    return b * _NPB + _OFF[lvl] + p


def _body(*refs):
    t_cls = refs[0:5]
    t_bbox = refs[5:10]
    t_angle = refs[10:15]
    t_ctr = refs[15:20]
    s_cls = refs[20:25]
    s_bbox = refs[25:30]
    s_angle = refs[30:35]
    s_ctr = refs[35:40]
    out = refs[40]
    sco = refs[41:46]

    for l in range(_NLVL):
        ts = jax.nn.sigmoid(t_cls[l][...])
        sco[l][...] = jnp.max(ts, axis=1, keepdims=True)

    def count_cmp(tf, strict):
        c = jnp.int32(0)
        for l in range(_NLVL):
            s = sco[l][...]
            pred = (s > tf) if strict else (s >= tf)
            c += jnp.sum(pred.astype(jnp.int32))
        return c

    def bs_val(i, carry):
        lo, hi = carry
        mid = (lo + hi) // 2
        tf = jax.lax.bitcast_convert_type(mid, jnp.float32)
        big = count_cmp(tf, False) >= _K
        return (jnp.where(big, mid, lo), jnp.where(big, hi, mid))

    lo, _ = jax.lax.fori_loop(
        0, 31, bs_val, (jnp.int32(0), jnp.int32(_ONE_BITS)))
    T = jax.lax.bitcast_convert_type(lo, jnp.float32)

    c_gt = count_cmp(T, True)
    m = _K - c_gt

    def count_tie_le(bound):
        c = jnp.int32(0)
        for l in range(_NLVL):
            s = sco[l][...]
            idx = _anchor_index(s.shape, l)
            c += jnp.sum(((s == T) & (idx <= bound)).astype(jnp.int32))
        return c

    def bs_idx(i, carry):
        lo2, hi2 = carry
        mid = (lo2 + hi2) // 2
        ok = count_tie_le(mid) >= m
        return (jnp.where(ok, lo2, mid), jnp.where(ok, mid, hi2))

    _, I = jax.lax.fori_loop(
        0, 17, bs_idx, (jnp.int32(-1), jnp.int32(_N - 1)))

    cls_sum = jnp.float32(0.0)
    bbox_sum = jnp.float32(0.0)
    ctr_sum = jnp.float32(0.0)
    fg_sum = jnp.float32(0.0)
    s_sum = jnp.float32(0.0)
    for l in range(_NLVL):
        s = sco[l][...]
        idx = _anchor_index(s.shape, l)
        mask = (s > T) | ((s == T) & (idx <= I))

        ts = jax.nn.sigmoid(t_cls[l][...])
        ps = jax.nn.sigmoid(s_cls[l][...])
        base = _bce(ps, jnp.zeros_like(ps)) * jnp.square(ps)
        masked = _bce(ps, ts) * jnp.square(ts - ps)
        cls_sum += jnp.sum(jnp.where(mask, masked, base))

        tc_ = jax.nn.sigmoid(t_ctr[l][...])
        d4 = jnp.sum(_smooth_l1(s_bbox[l][...], t_bbox[l][...]),
                     axis=1, keepdims=True)
        da = _smooth_l1(s_angle[l][...], t_angle[l][...])
        bbox_sum += jnp.sum(jnp.where(mask, (d4 + da) * tc_, 0.0))

        sc_ = jax.nn.sigmoid(s_ctr[l][...])
        ctr_sum += jnp.sum(jnp.where(mask, _bce(sc_, tc_), 0.0))

        fg_sum += jnp.sum(jnp.where(s > T, s, 0.0))
        s_sum += jnp.sum(s)

    fg_num = fg_sum + m.astype(jnp.float32) * T
    out[0] = cls_sum / fg_num
    out[1] = bbox_sum / jnp.float32(_K * 5)
    out[2] = ctr_sum / jnp.float32(_K)
    out[3] = s_sum / jnp.float32(_N)


def kernel(t_cls_0, t_bbox_0, t_angle_0, t_ctr_0,
           t_cls_1, t_bbox_1, t_angle_1, t_ctr_1,
           t_cls_2, t_bbox_2, t_angle_2, t_ctr_2,
           t_cls_3, t_bbox_3, t_angle_3, t_ctr_3,
           t_cls_4, t_bbox_4, t_angle_4, t_ctr_4,
           s_cls_0, s_bbox_0, s_angle_0, s_ctr_0,
           s_cls_1, s_bbox_1, s_angle_1, s_ctr_1,
           s_cls_2, s_bbox_2, s_angle_2, s_ctr_2,
           s_cls_3, s_bbox_3, s_angle_3, s_ctr_3,
           s_cls_4, s_bbox_4, s_angle_4, s_ctr_4):
    inp = dict(locals())
    args = []
    for pre in ('t', 's'):
        for nm in ('cls', 'bbox', 'angle', 'ctr'):
            for l in range(_NLVL):
                x = inp['%s_%s_%d' % (pre, nm, l)]
                args.append(x.reshape(_B, x.shape[1], _HH[l]))

    out = pl.pallas_call(
        _body,
        out_shape=jax.ShapeDtypeStruct((4,), jnp.float32),
        out_specs=pl.BlockSpec(memory_space=pltpu.SMEM),
        scratch_shapes=[pltpu.VMEM((_B, 1, hh), jnp.float32) for hh in _HH],
    )(*args)
    return (out[0], out[1], out[2], out[3])


# speedup vs baseline: 2.9010x; 1.2439x over previous
import jax
import jax.numpy as jnp
from jax.experimental import pallas as pl
from jax.experimental.pallas import tpu as pltpu

_B = 16
_CLS = 16
_HS = (64, 32, 16, 8, 4)
_HH = tuple(h * h for h in _HS)
_NPB = sum(_HH)
_N = _B * _NPB
_K = max(int(_N * 0.01), 2)
_OFF = (0, 4096, 5120, 5376, 5440)
_NLVL = 5
_ONE_BITS = 0x3F800001
_ROWS = (32, 8, 2, 1, 1)
_LANES = (128, 128, 128, 64, 16)
_ROFF = (0, 512, 640, 672, 688)
_FROWS = 704


def _bce(p, t):
    p = jnp.clip(p, 1e-12, 1.0 - 1e-12)
    return -(t * jnp.log(p) + (1.0 - t) * jnp.log1p(-p))


def _smooth_l1(x, y):
    d = jnp.abs(x - y)
    return jnp.where(d < 1.0, 0.5 * d * d, d - 0.5)


def _global_index(lvl):
    shape = (_B, _ROWS[lvl], 128)
    b = jax.lax.broadcasted_iota(jnp.int32, shape, 0)
    r = jax.lax.broadcasted_iota(jnp.int32, shape, 1)
    lane = jax.lax.broadcasted_iota(jnp.int32, shape, 2)
    idx = b * _NPB + _OFF[lvl] + r * 128 + lane
    if _LANES[lvl] < 128:
        idx = jnp.where(lane < _LANES[lvl], idx, _N)
    return idx


def _pad_lanes(x, lvl, fill):
    if _LANES[lvl] == 128:
        return x
    pad = jnp.full((_B, _ROWS[lvl], 128 - _LANES[lvl]), fill, x.dtype)
    return jnp.concatenate([x, pad], axis=2)


def _body(*refs):
    t_cls = refs[0:5]
    t_bbox = refs[5:10]
    t_angle = refs[10:15]
    t_ctr = refs[15:20]
    s_cls = refs[20:25]
    s_bbox = refs[25:30]
    s_angle = refs[30:35]
    s_ctr = refs[35:40]
    out = refs[40]
    f_s, f_dl, f_bb, f_ct, f_ix = refs[41:46]

    base_total = jnp.float32(0.0)
    s_sum = jnp.float32(0.0)
    for l in range(_NLVL):
        ro, nr = _ROFF[l], _B * _ROWS[l]
        sl = slice(ro, ro + nr)

        ts = jax.nn.sigmoid(t_cls[l][...])
        ps = jax.nn.sigmoid(s_cls[l][...])
        s = jnp.max(ts, axis=1)
        s_sum += jnp.sum(s)
        f_s[sl] = _pad_lanes(s, l, -1.0).reshape(nr, 128)
        f_ix[sl] = _global_index(l).reshape(nr, 128)

        pc = jnp.clip(ps, 1e-12, 1.0 - 1e-12)
        lp = jnp.log(pc)
        l1p = jnp.log1p(-pc)
        base = -l1p * jnp.square(ps)
        masked = -(ts * lp + (1.0 - ts) * l1p) * jnp.square(ts - ps)
        base_total += jnp.sum(base)
        delta = jnp.sum(masked - base, axis=1)
        f_dl[sl] = _pad_lanes(delta, l, 0.0).reshape(nr, 128)

        tc_ = jax.nn.sigmoid(t_ctr[l][...])[:, 0]
        sc_ = jax.nn.sigmoid(s_ctr[l][...])[:, 0]
        d4 = jnp.sum(_smooth_l1(s_bbox[l][...], t_bbox[l][...]), axis=1)
        da = _smooth_l1(s_angle[l][...], t_angle[l][...])[:, 0]
        f_bb[sl] = _pad_lanes((d4 + da) * tc_, l, 0.0).reshape(nr, 128)
        f_ct[sl] = _pad_lanes(_bce(sc_, tc_), l, 0.0).reshape(nr, 128)

    def count_ge(tf):
        return jnp.sum((f_s[...] >= tf).astype(jnp.int32))

    def bs_val(i, carry):
        lo, hi = carry
        mid = (lo + hi) // 2
        tf = jax.lax.bitcast_convert_type(mid, jnp.float32)
        big = count_ge(tf) >= _K
        return (jnp.where(big, mid, lo), jnp.where(big, hi, mid))

    lo, _ = jax.lax.fori_loop(
        0, 31, bs_val, (jnp.int32(0), jnp.int32(_ONE_BITS)))
    T = jax.lax.bitcast_convert_type(lo, jnp.float32)

    c_gt = jnp.sum((f_s[...] > T).astype(jnp.int32))
    m = _K - c_gt

    def bs_idx(i, carry):
        lo2, hi2 = carry
        mid = (lo2 + hi2) // 2
        c = jnp.sum(((f_s[...] == T) & (f_ix[...] <= mid)).astype(jnp.int32))
        ok = c >= m
        return (jnp.where(ok, lo2, mid), jnp.where(ok, mid, hi2))

    _, I = jax.lax.fori_loop(
        0, 17, bs_idx, (jnp.int32(-1), jnp.int32(_N - 1)))

    s = f_s[...]
    mask = (s > T) | ((s == T) & (f_ix[...] <= I))
    cls_sum = base_total + jnp.sum(jnp.where(mask, f_dl[...], 0.0))
    bbox_sum = jnp.sum(jnp.where(mask, f_bb[...], 0.0))
    ctr_sum = jnp.sum(jnp.where(mask, f_ct[...], 0.0))
    fg_num = jnp.sum(jnp.where(s > T, s, 0.0)) + m.astype(jnp.float32) * T

    out[0] = cls_sum / fg_num
    out[1] = bbox_sum / jnp.float32(_K * 5)
    out[2] = ctr_sum / jnp.float32(_K)
    out[3] = s_sum / jnp.float32(_N)


def kernel(t_cls_0, t_bbox_0, t_angle_0, t_ctr_0,
           t_cls_1, t_bbox_1, t_angle_1, t_ctr_1,
           t_cls_2, t_bbox_2, t_angle_2, t_ctr_2,
           t_cls_3, t_bbox_3, t_angle_3, t_ctr_3,
           t_cls_4, t_bbox_4, t_angle_4, t_ctr_4,
           s_cls_0, s_bbox_0, s_angle_0, s_ctr_0,
           s_cls_1, s_bbox_1, s_angle_1, s_ctr_1,
           s_cls_2, s_bbox_2, s_angle_2, s_ctr_2,
           s_cls_3, s_bbox_3, s_angle_3, s_ctr_3,
           s_cls_4, s_bbox_4, s_angle_4, s_ctr_4):
    inp = dict(locals())
    args = []
    for pre in ('t', 's'):
        for nm in ('cls', 'bbox', 'angle', 'ctr'):
            for l in range(_NLVL):
                x = inp['%s_%s_%d' % (pre, nm, l)]
                args.append(x.reshape(_B, x.shape[1], _ROWS[l], _LANES[l]))

    out = pl.pallas_call(
        _body,
        out_shape=jax.ShapeDtypeStruct((4,), jnp.float32),
        out_specs=pl.BlockSpec(memory_space=pltpu.SMEM),
        scratch_shapes=[pltpu.VMEM((_FROWS, 128), jnp.float32)] * 4
        + [pltpu.VMEM((_FROWS, 128), jnp.int32)],
    )(*args)
    return (out[0], out[1], out[2], out[3])


# speedup vs baseline: 3.4844x; 1.2011x over previous
import jax
import jax.numpy as jnp
from jax.experimental import pallas as pl
from jax.experimental.pallas import tpu as pltpu

_B = 16
_CLS = 16
_HS = (64, 32, 16, 8, 4)
_HH = tuple(h * h for h in _HS)
_NPB = sum(_HH)
_N = _B * _NPB
_K = max(int(_N * 0.01), 2)
_OFF = (0, 4096, 5120, 5376, 5440)
_NLVL = 5
_ONE_BITS = 0x3F800001
_ROWS = (32, 8, 2, 1, 1)
_LANES = (128, 128, 128, 64, 16)
_ROFF = (0, 512, 640, 672, 688)
_FROWS = 704


def _bce(p, t):
    p = jnp.clip(p, 1e-12, 1.0 - 1e-12)
    return -(t * jnp.log(p) + (1.0 - t) * jnp.log1p(-p))


def _smooth_l1(x, y):
    d = jnp.abs(x - y)
    return jnp.where(d < 1.0, 0.5 * d * d, d - 0.5)


def _global_index(lvl):
    shape = (_B, _ROWS[lvl], 128)
    b = jax.lax.broadcasted_iota(jnp.int32, shape, 0)
    r = jax.lax.broadcasted_iota(jnp.int32, shape, 1)
    lane = jax.lax.broadcasted_iota(jnp.int32, shape, 2)
    idx = b * _NPB + _OFF[lvl] + r * 128 + lane
    if _LANES[lvl] < 128:
        idx = jnp.where(lane < _LANES[lvl], idx, _N)
    return idx


def _pad_lanes(x, lvl, fill):
    if _LANES[lvl] == 128:
        return x
    pad = jnp.full((_B, _ROWS[lvl], 128 - _LANES[lvl]), fill, x.dtype)
    return jnp.concatenate([x, pad], axis=2)


def _body(*refs):
    lvl_refs = refs[0:5]
    out = refs[5]
    f_s, f_dl, f_bb, f_ct, f_ix = refs[6:11]
    t_cls = [r.at[:, 0:16] for r in lvl_refs]
    t_bbox = [r.at[:, 16:20] for r in lvl_refs]
    t_angle = [r.at[:, 20:21] for r in lvl_refs]
    t_ctr = [r.at[:, 21:22] for r in lvl_refs]
    s_cls = [r.at[:, 22:38] for r in lvl_refs]
    s_bbox = [r.at[:, 38:42] for r in lvl_refs]
    s_angle = [r.at[:, 42:43] for r in lvl_refs]
    s_ctr = [r.at[:, 43:44] for r in lvl_refs]

    base_total = jnp.float32(0.0)
    s_sum = jnp.float32(0.0)
    for l in range(_NLVL):
        ro, nr = _ROFF[l], _B * _ROWS[l]
        sl = slice(ro, ro + nr)

        ts = jax.nn.sigmoid(t_cls[l][...])
        ps = jax.nn.sigmoid(s_cls[l][...])
        s = jnp.max(ts, axis=1)
        s_sum += jnp.sum(s)
        f_s[sl] = _pad_lanes(s, l, -1.0).reshape(nr, 128)
        f_ix[sl] = _global_index(l).reshape(nr, 128)

        pc = jnp.clip(ps, 1e-12, 1.0 - 1e-12)
        lp = jnp.log(pc)
        l1p = jnp.log1p(-pc)
        base = -l1p * jnp.square(ps)
        masked = -(ts * lp + (1.0 - ts) * l1p) * jnp.square(ts - ps)
        base_total += jnp.sum(base)
        delta = jnp.sum(masked - base, axis=1)
        f_dl[sl] = _pad_lanes(delta, l, 0.0).reshape(nr, 128)

        tc_ = jax.nn.sigmoid(t_ctr[l][...])[:, 0]
        sc_ = jax.nn.sigmoid(s_ctr[l][...])[:, 0]
        d4 = jnp.sum(_smooth_l1(s_bbox[l][...], t_bbox[l][...]), axis=1)
        da = _smooth_l1(s_angle[l][...], t_angle[l][...])[:, 0]
        f_bb[sl] = _pad_lanes((d4 + da) * tc_, l, 0.0).reshape(nr, 128)
        f_ct[sl] = _pad_lanes(_bce(sc_, tc_), l, 0.0).reshape(nr, 128)

    def count_ge(tf):
        return jnp.sum((f_s[...] >= tf).astype(jnp.int32))

    def bs_val(i, carry):
        lo, hi = carry
        mid = (lo + hi) // 2
        tf = jax.lax.bitcast_convert_type(mid, jnp.float32)
        big = count_ge(tf) >= _K
        return (jnp.where(big, mid, lo), jnp.where(big, hi, mid))

    lo, _ = jax.lax.fori_loop(
        0, 31, bs_val, (jnp.int32(0), jnp.int32(_ONE_BITS)))
    T = jax.lax.bitcast_convert_type(lo, jnp.float32)

    c_gt = jnp.sum((f_s[...] > T).astype(jnp.int32))
    m = _K - c_gt

    def bs_idx(i, carry):
        lo2, hi2 = carry
        mid = (lo2 + hi2) // 2
        c = jnp.sum(((f_s[...] == T) & (f_ix[...] <= mid)).astype(jnp.int32))
        ok = c >= m
        return (jnp.where(ok, lo2, mid), jnp.where(ok, mid, hi2))

    _, I = jax.lax.fori_loop(
        0, 17, bs_idx, (jnp.int32(-1), jnp.int32(_N - 1)))

    s = f_s[...]
    mask = (s > T) | ((s == T) & (f_ix[...] <= I))
    cls_sum = base_total + jnp.sum(jnp.where(mask, f_dl[...], 0.0))
    bbox_sum = jnp.sum(jnp.where(mask, f_bb[...], 0.0))
    ctr_sum = jnp.sum(jnp.where(mask, f_ct[...], 0.0))
    fg_num = jnp.sum(jnp.where(s > T, s, 0.0)) + m.astype(jnp.float32) * T

    out[0] = cls_sum / fg_num
    out[1] = bbox_sum / jnp.float32(_K * 5)
    out[2] = ctr_sum / jnp.float32(_K)
    out[3] = s_sum / jnp.float32(_N)


def kernel(t_cls_0, t_bbox_0, t_angle_0, t_ctr_0,
           t_cls_1, t_bbox_1, t_angle_1, t_ctr_1,
           t_cls_2, t_bbox_2, t_angle_2, t_ctr_2,
           t_cls_3, t_bbox_3, t_angle_3, t_ctr_3,
           t_cls_4, t_bbox_4, t_angle_4, t_ctr_4,
           s_cls_0, s_bbox_0, s_angle_0, s_ctr_0,
           s_cls_1, s_bbox_1, s_angle_1, s_ctr_1,
           s_cls_2, s_bbox_2, s_angle_2, s_ctr_2,
           s_cls_3, s_bbox_3, s_angle_3, s_ctr_3,
           s_cls_4, s_bbox_4, s_angle_4, s_ctr_4):
    inp = dict(locals())
    args = []
    for l in range(_NLVL):
        parts = []
        for pre in ('t', 's'):
            for nm in ('cls', 'bbox', 'angle', 'ctr'):
                x = inp['%s_%s_%d' % (pre, nm, l)]
                parts.append(x.reshape(_B, x.shape[1], _ROWS[l], _LANES[l]))
        args.append(jnp.concatenate(parts, axis=1))

    out = pl.pallas_call(
        _body,
        out_shape=jax.ShapeDtypeStruct((4,), jnp.float32),
        out_specs=pl.BlockSpec(memory_space=pltpu.SMEM),
        scratch_shapes=[pltpu.VMEM((_FROWS, 128), jnp.float32)] * 4
        + [pltpu.VMEM((_FROWS, 128), jnp.int32)],
    )(*args)
    return (out[0], out[1], out[2], out[3])


# speedup vs baseline: 4.0009x; 1.1482x over previous
import jax
import jax.numpy as jnp
from jax.experimental import pallas as pl
from jax.experimental.pallas import tpu as pltpu

_B = 16
_CLS = 16
_HS = (64, 32, 16, 8, 4)
_HH = tuple(h * h for h in _HS)
_NPB = sum(_HH)
_N = _B * _NPB
_K = max(int(_N * 0.01), 2)
_OFF = (0, 4096, 5120, 5376, 5440)
_NLVL = 5
_ONE_BITS = 0x3F800001


def _bce(p, t):
    p = jnp.clip(p, 1e-12, 1.0 - 1e-12)
    return -(t * jnp.log(p) + (1.0 - t) * jnp.log1p(-p))


def _smooth_l1(x, y):
    d = jnp.abs(x - y)
    return jnp.where(d < 1.0, 0.5 * d * d, d - 0.5)


def _global_index(lvl):
    h = _HS[lvl]
    shape = (_B, h, h)
    b = jax.lax.broadcasted_iota(jnp.int32, shape, 0)
    y = jax.lax.broadcasted_iota(jnp.int32, shape, 1)
    x = jax.lax.broadcasted_iota(jnp.int32, shape, 2)
    return b * _NPB + _OFF[lvl] + y * h + x


def _body(*refs):
    t_cls = refs[0:5]
    t_bbox = refs[5:10]
    t_angle = refs[10:15]
    t_ctr = refs[15:20]
    s_cls = refs[20:25]
    s_bbox = refs[25:30]
    s_angle = refs[30:35]
    s_ctr = refs[35:40]
    out = refs[40]
    f_s = refs[41:46]
    f_dl = refs[46:51]
    f_bb = refs[51:56]
    f_ct = refs[56:61]
    acc = refs[61]

    b = pl.program_id(0)

    @pl.when(b == 0)
    def _init():
        acc[0] = jnp.float32(0.0)
        acc[1] = jnp.float32(0.0)

    base_total = jnp.float32(0.0)
    s_sum = jnp.float32(0.0)
    for l in range(_NLVL):
        h = _HS[l]
        ts = jax.nn.sigmoid(t_cls[l][0])
        ps = jax.nn.sigmoid(s_cls[l][0])
        s = jnp.max(ts, axis=0)
        s_sum += jnp.sum(s)
        f_s[l][pl.ds(b, 1)] = s.reshape(1, h, h)

        pc = jnp.clip(ps, 1e-12, 1.0 - 1e-12)
        lp = jnp.log(pc)
        l1p = jnp.log1p(-pc)
        base = -l1p * jnp.square(ps)
        masked = -(ts * lp + (1.0 - ts) * l1p) * jnp.square(ts - ps)
        base_total += jnp.sum(base)
        f_dl[l][pl.ds(b, 1)] = jnp.sum(masked - base, axis=0).reshape(1, h, h)

        tc_ = jax.nn.sigmoid(t_ctr[l][0])[0]
        sc_ = jax.nn.sigmoid(s_ctr[l][0])[0]
        d4 = jnp.sum(_smooth_l1(s_bbox[l][0], t_bbox[l][0]), axis=0)
        da = _smooth_l1(s_angle[l][0], t_angle[l][0])[0]
        f_bb[l][pl.ds(b, 1)] = ((d4 + da) * tc_).reshape(1, h, h)
        f_ct[l][pl.ds(b, 1)] = _bce(sc_, tc_).reshape(1, h, h)

    acc[0] += base_total
    acc[1] += s_sum

    @pl.when(b == _B - 1)
    def _finish():
        _tail(refs)


def _tail(refs):
    out = refs[40]
    f_s = refs[41:46]
    f_dl = refs[46:51]
    f_bb = refs[51:56]
    f_ct = refs[56:61]
    acc = refs[61]
    base_total = acc[0]
    s_sum = acc[1]

    def count_ge(tf, strict):
        c = jnp.int32(0)
        for l in range(_NLVL):
            s = f_s[l][...]
            pred = (s > tf) if strict else (s >= tf)
            c += jnp.sum(pred.astype(jnp.int32))
        return c

    def bs_val(i, carry):
        lo, hi = carry
        mid = (lo + hi) // 2
        tf = jax.lax.bitcast_convert_type(mid, jnp.float32)
        big = count_ge(tf, False) >= _K
        return (jnp.where(big, mid, lo), jnp.where(big, hi, mid))

    lo, _ = jax.lax.fori_loop(
        0, 31, bs_val, (jnp.int32(0), jnp.int32(_ONE_BITS)))
    T = jax.lax.bitcast_convert_type(lo, jnp.float32)

    c_gt = count_ge(T, True)
    m = _K - c_gt

    def count_tie_le(bound):
        c = jnp.int32(0)
        for l in range(_NLVL):
            s = f_s[l][...]
            idx = _global_index(l)
            c += jnp.sum(((s == T) & (idx <= bound)).astype(jnp.int32))
        return c

    def bs_idx(i, carry):
        lo2, hi2 = carry
        mid = (lo2 + hi2) // 2
        ok = count_tie_le(mid) >= m
        return (jnp.where(ok, lo2, mid), jnp.where(ok, mid, hi2))

    _, I = jax.lax.fori_loop(
        0, 17, bs_idx, (jnp.int32(-1), jnp.int32(_N - 1)))

    cls_sum = base_total
    bbox_sum = jnp.float32(0.0)
    ctr_sum = jnp.float32(0.0)
    fg_sum = jnp.float32(0.0)
    for l in range(_NLVL):
        s = f_s[l][...]
        mask = (s > T) | ((s == T) & (_global_index(l) <= I))
        cls_sum += jnp.sum(jnp.where(mask, f_dl[l][...], 0.0))
        bbox_sum += jnp.sum(jnp.where(mask, f_bb[l][...], 0.0))
        ctr_sum += jnp.sum(jnp.where(mask, f_ct[l][...], 0.0))
        fg_sum += jnp.sum(jnp.where(s > T, s, 0.0))
    fg_num = fg_sum + m.astype(jnp.float32) * T

    out[0] = cls_sum / fg_num
    out[1] = bbox_sum / jnp.float32(_K * 5)
    out[2] = ctr_sum / jnp.float32(_K)
    out[3] = s_sum / jnp.float32(_N)


def kernel(t_cls_0, t_bbox_0, t_angle_0, t_ctr_0,
           t_cls_1, t_bbox_1, t_angle_1, t_ctr_1,
           t_cls_2, t_bbox_2, t_angle_2, t_ctr_2,
           t_cls_3, t_bbox_3, t_angle_3, t_ctr_3,
           t_cls_4, t_bbox_4, t_angle_4, t_ctr_4,
           s_cls_0, s_bbox_0, s_angle_0, s_ctr_0,
           s_cls_1, s_bbox_1, s_angle_1, s_ctr_1,
           s_cls_2, s_bbox_2, s_angle_2, s_ctr_2,
           s_cls_3, s_bbox_3, s_angle_3, s_ctr_3,
           s_cls_4, s_bbox_4, s_angle_4, s_ctr_4):
    inp = dict(locals())
    args = []
    in_specs = []
    for pre in ('t', 's'):
        for nm in ('cls', 'bbox', 'angle', 'ctr'):
            for l in range(_NLVL):
                x = inp['%s_%s_%d' % (pre, nm, l)]
                args.append(x)
                in_specs.append(pl.BlockSpec(
                    (1,) + x.shape[1:], lambda b: (b, 0, 0, 0)))

    sco = [pltpu.VMEM((_B, h, h), jnp.float32) for h in _HS]
    out = pl.pallas_call(
        _body,
        grid=(_B,),
        in_specs=in_specs,
        out_shape=jax.ShapeDtypeStruct((4,), jnp.float32),
        out_specs=pl.BlockSpec(memory_space=pltpu.SMEM),
        scratch_shapes=sco * 4 + [pltpu.SMEM((2,), jnp.float32)],
    )(*args)
    return (out[0], out[1], out[2], out[3])


# speedup vs baseline: 4.5142x; 1.1283x over previous
import jax
import jax.numpy as jnp
from jax.experimental import pallas as pl
from jax.experimental.pallas import tpu as pltpu

_B = 16
_CLS = 16
_HS = (64, 32, 16, 8, 4)
_HH = tuple(h * h for h in _HS)
_NPB = sum(_HH)
_N = _B * _NPB
_K = max(int(_N * 0.01), 2)
_OFF = (0, 4096, 5120, 5376, 5440)
_NLVL = 5
_ONE_BITS = 0x3F800001
_FROWS = 1024
_ROW0 = (0, 0, 512, 768, 896)
_RSTEP = (64, 32, 16, 8, 8)
_LANE0 = (0, 64, 64, 64, 64)


def _smooth_l1(x, y):
    d = jnp.abs(x - y)
    return jnp.where(d < 1.0, 0.5 * d * d, d - 0.5)


def _level_index(lvl, b):
    h = _HS[lvl]
    y = jax.lax.broadcasted_iota(jnp.int32, (h, h), 0)
    x = jax.lax.broadcasted_iota(jnp.int32, (h, h), 1)
    return b * _NPB + _OFF[lvl] + y * h + x


def _body(*refs):
    t_cls = refs[0:5]
    t_bbox = refs[5:10]
    t_angle = refs[10:15]
    t_ctr = refs[15:20]
    s_cls = refs[20:25]
    s_bbox = refs[25:30]
    s_angle = refs[30:35]
    s_ctr = refs[35:40]
    out = refs[40]
    f_s, f_dl, f_bb, f_ct, f_ix, acc = refs[41:47]

    b = pl.program_id(0)

    @pl.when(b == 0)
    def _init():
        acc[0] = jnp.float32(0.0)
        acc[1] = jnp.float32(0.0)
        f_s[...] = jnp.full((_FROWS, 128), -1.0, jnp.float32)
        f_ix[...] = jnp.full((_FROWS, 128), _N, jnp.int32)

    base_total = jnp.float32(0.0)
    s_sum = jnp.float32(0.0)
    for l in range(_NLVL):
        h = _HS[l]
        r0 = _ROW0[l] + b * _RSTEP[l]
        rs = pl.ds(r0, h)
        ls = slice(_LANE0[l], _LANE0[l] + h)

        x = s_cls[l][0]
        ts = jax.nn.sigmoid(t_cls[l][0])
        ps = jax.nn.sigmoid(x)
        s = jnp.max(ts, axis=0)
        s_sum += jnp.sum(s)
        f_s[rs, ls] = s
        f_ix[rs, ls] = _level_index(l, b)

        l1p = jnp.log1p(-ps)
        base_total += jnp.sum(-l1p * jnp.square(ps))
        d = ts - ps
        delta = -ts * (x * jnp.square(d) + l1p * (d - ps))
        f_dl[rs, ls] = jnp.sum(delta, axis=0)

        tc_ = jax.nn.sigmoid(t_ctr[l][0])[0]
        sc_ = jax.nn.sigmoid(s_ctr[l][0])[0]
        d4 = jnp.sum(_smooth_l1(s_bbox[l][0], t_bbox[l][0]), axis=0)
        da = _smooth_l1(s_angle[l][0], t_angle[l][0])[0]
        f_bb[rs, ls] = (d4 + da) * tc_
        pcc = jnp.clip(sc_, 1e-12, 1.0 - 1e-12)
        f_ct[rs, ls] = -(tc_ * jnp.log(pcc) + (1.0 - tc_) * jnp.log1p(-pcc))

    acc[0] += base_total
    acc[1] += s_sum

    @pl.when(b == _B - 1)
    def _finish():
        _tail(out, f_s, f_dl, f_bb, f_ct, f_ix, acc)


def _tail(out, f_s, f_dl, f_bb, f_ct, f_ix, acc):
    base_total = acc[0]
    s_sum = acc[1]

    def bs_val(i, carry):
        lo, hi = carry
        mid = (lo + hi) // 2
        tf = jax.lax.bitcast_convert_type(mid, jnp.float32)
        big = jnp.sum((f_s[...] >= tf).astype(jnp.int32)) >= _K
        return (jnp.where(big, mid, lo), jnp.where(big, hi, mid))

    lo, _ = jax.lax.fori_loop(
        0, 31, bs_val, (jnp.int32(0), jnp.int32(_ONE_BITS)))
    T = jax.lax.bitcast_convert_type(lo, jnp.float32)

    s = f_s[...]
    ix = f_ix[...]
    c_gt = jnp.sum((s > T).astype(jnp.int32))
    m = _K - c_gt

    def bs_idx(i, carry):
        lo2, hi2 = carry
        mid = (lo2 + hi2) // 2
        c = jnp.sum(((f_s[...] == T) & (f_ix[...] <= mid)).astype(jnp.int32))
        ok = c >= m
        return (jnp.where(ok, lo2, mid), jnp.where(ok, mid, hi2))

    _, I = jax.lax.fori_loop(
        0, 17, bs_idx, (jnp.int32(-1), jnp.int32(_N - 1)))

    mask = (s > T) | ((s == T) & (ix <= I))
    cls_sum = base_total + jnp.sum(jnp.where(mask, f_dl[...], 0.0))
    bbox_sum = jnp.sum(jnp.where(mask, f_bb[...], 0.0))
    ctr_sum = jnp.sum(jnp.where(mask, f_ct[...], 0.0))
    fg_num = jnp.sum(jnp.where(s > T, s, 0.0)) + m.astype(jnp.float32) * T

    out[0] = cls_sum / fg_num
    out[1] = bbox_sum / jnp.float32(_K * 5)
    out[2] = ctr_sum / jnp.float32(_K)
    out[3] = s_sum / jnp.float32(_N)


def kernel(t_cls_0, t_bbox_0, t_angle_0, t_ctr_0,
           t_cls_1, t_bbox_1, t_angle_1, t_ctr_1,
           t_cls_2, t_bbox_2, t_angle_2, t_ctr_2,
           t_cls_3, t_bbox_3, t_angle_3, t_ctr_3,
           t_cls_4, t_bbox_4, t_angle_4, t_ctr_4,
           s_cls_0, s_bbox_0, s_angle_0, s_ctr_0,
           s_cls_1, s_bbox_1, s_angle_1, s_ctr_1,
           s_cls_2, s_bbox_2, s_angle_2, s_ctr_2,
           s_cls_3, s_bbox_3, s_angle_3, s_ctr_3,
           s_cls_4, s_bbox_4, s_angle_4, s_ctr_4):
    inp = dict(locals())
    args = []
    in_specs = []
    for pre in ('t', 's'):
        for nm in ('cls', 'bbox', 'angle', 'ctr'):
            for l in range(_NLVL):
                x = inp['%s_%s_%d' % (pre, nm, l)]
                args.append(x)
                in_specs.append(pl.BlockSpec(
                    (1,) + x.shape[1:], lambda b: (b, 0, 0, 0)))

    out = pl.pallas_call(
        _body,
        grid=(_B,),
        in_specs=in_specs,
        out_shape=jax.ShapeDtypeStruct((4,), jnp.float32),
        out_specs=pl.BlockSpec(memory_space=pltpu.SMEM),
        scratch_shapes=[pltpu.VMEM((_FROWS, 128), jnp.float32)] * 4
        + [pltpu.VMEM((_FROWS, 128), jnp.int32),
           pltpu.SMEM((2,), jnp.float32)],
    )(*args)
    return (out[0], out[1], out[2], out[3])
